# baseline (speedup 1.0000x reference)
import jax
import jax.numpy as jnp
from jax.experimental import pallas as pl
from jax.experimental.pallas import tpu as pltpu

B = 1024
LAT = 64
ACTIONS = 64
POL_CH = 2


def _body(x_ref, wcb_ref, bcb_ref, wfc_ref, bfc_ref, out_ref):
    tiles = []
    for o in range(POL_CH):
        for h in range(16):
            for wh in range(2):
                def cbody(c, acc, h=h, wh=wh, o=o):
                    return acc + x_ref[c, h, wh, 0] * wcb_ref[o, c]
                acc = jax.lax.fori_loop(
                    0, LAT, cbody, jnp.zeros((8, 128), jnp.float32),
                    unroll=16,
                )
                tiles.append(jnp.maximum(acc + bcb_ref[o], 0.0))
    rhs = jnp.concatenate(tiles, axis=0)
    out_ref[...] = (
        jnp.dot(wfc_ref[...], rhs, preferred_element_type=jnp.float32)
        + bfc_ref[...]
    )


def kernel(x, W_conv, b_conv, W_fc, b_fc):
    x6 = x.reshape(8, 128, LAT, 16, 2, 8).transpose(2, 3, 4, 0, 5, 1)
    wcb = jnp.broadcast_to(W_conv[:, :, None, None], (POL_CH, LAT, 8, 128))
    bcb = jnp.broadcast_to(b_conv[:, None, None], (POL_CH, 8, 128))
    bfc_col = b_fc[:, None]

    out = pl.pallas_call(
        _body,
        grid=(8,),
        in_specs=[
            pl.BlockSpec((LAT, 16, 2, 1, 8, 128), lambda i: (0, 0, 0, i, 0, 0)),
            pl.BlockSpec((POL_CH, LAT, 8, 128), lambda i: (0, 0, 0, 0)),
            pl.BlockSpec((POL_CH, 8, 128), lambda i: (0, 0, 0)),
            pl.BlockSpec((ACTIONS, POL_CH * 256), lambda i: (0, 0)),
            pl.BlockSpec((ACTIONS, 1), lambda i: (0, 0)),
        ],
        out_specs=pl.BlockSpec((ACTIONS, 128), lambda i: (0, i)),
        out_shape=jax.ShapeDtypeStruct((ACTIONS, B), jnp.float32),
        compiler_params=pltpu.CompilerParams(
            dimension_semantics=("arbitrary",),
        ),
    )(x6, wcb, bcb, W_fc, bfc_col)
    return out.T


# speedup vs baseline: 1.6292x; 1.6292x over previous
import jax
import jax.numpy as jnp
from jax.experimental import pallas as pl
from jax.experimental.pallas import tpu as pltpu

B = 1024
LAT = 64
ACTIONS = 64
POL_CH = 2


def _body(x_ref, wcb_ref, bcb_ref, wfc_ref, bfc_ref, out_ref):
    tiles = [None] * (POL_CH * 32)
    for h in range(16):
        for wh in range(2):
            xs = x_ref[:, h, wh, 0]
            for o in range(POL_CH):
                acc = jnp.sum(xs * wcb_ref[o], axis=0)
                tiles[(o * 16 + h) * 2 + wh] = jnp.maximum(
                    acc + bcb_ref[o], 0.0
                )
    rhs = jnp.concatenate(tiles, axis=0)
    out_ref[...] = (
        jnp.dot(wfc_ref[...], rhs, preferred_element_type=jnp.float32)
        + bfc_ref[...]
    )


def kernel(x, W_conv, b_conv, W_fc, b_fc):
    x6 = x.reshape(8, 128, LAT, 16, 2, 8).transpose(2, 3, 4, 0, 5, 1)
    wcb = jnp.broadcast_to(W_conv[:, :, None, None], (POL_CH, LAT, 8, 128))
    bcb = jnp.broadcast_to(b_conv[:, None, None], (POL_CH, 8, 128))
    bfc_col = b_fc[:, None]

    out = pl.pallas_call(
        _body,
        grid=(8,),
        in_specs=[
            pl.BlockSpec((LAT, 16, 2, 1, 8, 128), lambda i: (0, 0, 0, i, 0, 0)),
            pl.BlockSpec((POL_CH, LAT, 8, 128), lambda i: (0, 0, 0, 0)),
            pl.BlockSpec((POL_CH, 8, 128), lambda i: (0, 0, 0)),
            pl.BlockSpec((ACTIONS, POL_CH * 256), lambda i: (0, 0)),
            pl.BlockSpec((ACTIONS, 1), lambda i: (0, 0)),
        ],
        out_specs=pl.BlockSpec((ACTIONS, 128), lambda i: (0, i)),
        out_shape=jax.ShapeDtypeStruct((ACTIONS, B), jnp.float32),
        compiler_params=pltpu.CompilerParams(
            dimension_semantics=("arbitrary",),
        ),
    )(x6, wcb, bcb, W_fc, bfc_col)
    return out.T


# speedup vs baseline: 1.8760x; 1.1515x over previous
import jax
import jax.numpy as jnp
from jax.experimental import pallas as pl
from jax.experimental.pallas import tpu as pltpu

B = 1024
LAT = 64
ACTIONS = 64
POL_CH = 2
NCHAIN = 4


def _body(x_ref, wc_ref, bcb_ref, wfc_ref, bfc_ref, out_ref):
    tiles = [None] * (POL_CH * 32)
    for h in range(16):
        for wh in range(2):
            a0 = [None] * NCHAIN
            a1 = [None] * NCHAIN
            for c in range(LAT):
                t = x_ref[c, h, wh, 0]
                p0 = t * wc_ref[0, c]
                p1 = t * wc_ref[1, c]
                k = c % NCHAIN
                a0[k] = p0 if a0[k] is None else a0[k] + p0
                a1[k] = p1 if a1[k] is None else a1[k] + p1
            s0 = (a0[0] + a0[1]) + (a0[2] + a0[3])
            s1 = (a1[0] + a1[1]) + (a1[2] + a1[3])
            tiles[h * 2 + wh] = jnp.maximum(s0 + bcb_ref[0], 0.0)
            tiles[32 + h * 2 + wh] = jnp.maximum(s1 + bcb_ref[1], 0.0)
    rhs = jnp.concatenate(tiles, axis=0)
    out_ref[...] = (
        jnp.dot(wfc_ref[...], rhs, preferred_element_type=jnp.float32)
        + bfc_ref[...]
    )


def kernel(x, W_conv, b_conv, W_fc, b_fc):
    x6 = x.reshape(8, 128, LAT, 16, 2, 8).transpose(2, 3, 4, 0, 5, 1)
    bcb = jnp.broadcast_to(b_conv[:, None, None], (POL_CH, 8, 128))
    bfc_col = b_fc[:, None]

    out = pl.pallas_call(
        _body,
        grid=(8,),
        in_specs=[
            pl.BlockSpec((LAT, 16, 2, 1, 8, 128), lambda i: (0, 0, 0, i, 0, 0)),
            pl.BlockSpec(memory_space=pltpu.SMEM),
            pl.BlockSpec((POL_CH, 8, 128), lambda i: (0, 0, 0)),
            pl.BlockSpec((ACTIONS, POL_CH * 256), lambda i: (0, 0)),
            pl.BlockSpec((ACTIONS, 1), lambda i: (0, 0)),
        ],
        out_specs=pl.BlockSpec((ACTIONS, 128), lambda i: (0, i)),
        out_shape=jax.ShapeDtypeStruct((ACTIONS, B), jnp.float32),
        compiler_params=pltpu.CompilerParams(
            dimension_semantics=("arbitrary",),
        ),
    )(x6, W_conv, bcb, W_fc, bfc_col)
    return out.T
